# baseline (speedup 1.0000x reference)
import functools

import jax
import jax.numpy as jnp
from jax import lax
from jax.experimental import pallas as pl
from jax.experimental.pallas import tpu as pltpu
from jax.experimental.pallas import tpu_sc as plsc

_NC = 2
_NS = 16
_LANES = 16
_CHUNK = 64


@functools.lru_cache(maxsize=None)
def _sc_masked_rev_copy(b, l, d, num_k):
    n_workers = _NC * _NS
    tiles_per_batch = n_workers // b
    rows_per_tile = l // tiles_per_batch
    n_chunks = rows_per_tile // _CHUNK
    vecs_per_row = d // _LANES
    mesh = plsc.VectorSubcoreMesh(
        core_axis_name="c", subcore_axis_name="s",
        num_cores=_NC, num_subcores=_NS)

    n_buf = 2

    def body(x2d, tkf, out2d, *scratch):
        tkv, mask_ref = scratch[0], scratch[1]
        gidxs = scratch[2:2 + n_buf]
        bufs = scratch[2 + n_buf:2 + 2 * n_buf]
        gsems = scratch[2 + 2 * n_buf:2 + 3 * n_buf]
        wsems = scratch[2 + 3 * n_buf:2 + 4 * n_buf]
        wid = lax.axis_index("s") * _NC + lax.axis_index("c")
        batch = wid // tiles_per_batch
        r0 = (wid % tiles_per_batch) * rows_per_tile

        iota16 = lax.iota(jnp.int32, _LANES)

        def issue_gather(c):
            p = c % n_buf
            top = batch * l + (l - 1) - r0 - c * _CHUNK
            gidx = gidxs[p]
            for g in range(_CHUNK // _LANES):
                gidx[pl.ds(g * _LANES, _LANES)] = top - g * _LANES - iota16
            return pltpu.async_copy(x2d.at[gidx], bufs[p], gsems[p])

        gdesc = [issue_gather(c) for c in range(n_buf - 1)] + [None]

        pltpu.sync_copy(tkf.at[pl.ds(batch * num_k, num_k)], tkv)

        for i in range(rows_per_tile // _LANES):
            mask_ref[pl.ds(i * _LANES, _LANES)] = jnp.zeros(
                (_LANES,), jnp.float32)

        def scan_topk(c, _):
            t16 = tkv[pl.ds(c * _LANES, _LANES)]
            rel = t16 - r0
            sel = (rel >= 0) & (rel < rows_per_tile)
            pos = jnp.clip(rel, 0, rows_per_tile - 1)
            plsc.store_scatter(mask_ref, [pos],
                               jnp.ones((_LANES,), jnp.float32), mask=sel)
            return 0

        lax.fori_loop(0, num_k // _LANES, scan_topk, 0)

        zeros16 = jnp.zeros((_LANES,), jnp.float32)

        def zero_unselected(c, buf):
            def zero_row(j, _):
                m = mask_ref[pl.ds(c * _CHUNK + j, _LANES)][0]

                @pl.when(m == 0.0)
                def _():
                    for g in range(vecs_per_row):
                        buf[j, pl.ds(g * _LANES, _LANES)] = zeros16

                return 0

            lax.fori_loop(0, _CHUNK, zero_row, 0)

        wdesc = {}
        waited = set()
        for c in range(n_chunks):
            p = c % n_buf
            gdesc[p].wait()
            zero_unselected(c, bufs[p])
            base = batch * l + r0 + c * _CHUNK
            wdesc[c] = pltpu.async_copy(bufs[p], out2d.at[pl.ds(base, _CHUNK)],
                                        wsems[p])
            nc = c + n_buf - 1
            if nc < n_chunks:
                prev_w = nc - n_buf
                if prev_w >= 0:
                    wdesc[prev_w].wait()
                    waited.add(prev_w)
                gdesc[nc % n_buf] = issue_gather(nc)
        for c in range(n_chunks):
            if c not in waited:
                wdesc[c].wait()

    return pl.kernel(
        body,
        out_type=jax.ShapeDtypeStruct((b * l, d), jnp.float32),
        mesh=mesh,
        compiler_params=pltpu.CompilerParams(needs_layout_passes=False),
        scratch_types=(
            [pltpu.VMEM((num_k,), jnp.int32),
             pltpu.VMEM((rows_per_tile + _LANES,), jnp.float32)]
            + [pltpu.VMEM((_CHUNK,), jnp.int32) for _ in range(2)]
            + [pltpu.VMEM((_CHUNK, d), jnp.float32) for _ in range(2)]
            + [pltpu.SemaphoreType.DMA for _ in range(4)]
        ),
    )


def kernel(x, select_x_mask, router_k, topk_index, Wq, Wk, Wv):
    b, l, d = x.shape
    num_k = topk_index.shape[1]
    x2d = x.reshape(b * l, d)
    tkf = topk_index.astype(jnp.int32).reshape(b * num_k)
    out2d = _sc_masked_rev_copy(b, l, d, num_k)(x2d, tkf)
    return out2d.reshape(b, l, d)


# speedup vs baseline: 1.2049x; 1.2049x over previous
import functools

import jax
import jax.numpy as jnp
from jax import lax
from jax.experimental import pallas as pl
from jax.experimental.pallas import tpu as pltpu
from jax.experimental.pallas import tpu_sc as plsc

_NC = 2
_NS = 16
_LANES = 16
_CHUNK = 32


@functools.lru_cache(maxsize=None)
def _sc_masked_rev_copy(b, l, d, num_k):
    n_workers = _NC * _NS
    tiles_per_batch = n_workers // b
    rows_per_tile = l // tiles_per_batch
    n_chunks = rows_per_tile // _CHUNK
    vecs_per_row = d // _LANES
    mesh = plsc.VectorSubcoreMesh(
        core_axis_name="c", subcore_axis_name="s",
        num_cores=_NC, num_subcores=_NS)

    n_buf = 5

    def body(x2d, tkf, out2d, *scratch):
        tkv, mask_ref = scratch[0], scratch[1]
        gidxs = scratch[2:2 + n_buf]
        bufs = scratch[2 + n_buf:2 + 2 * n_buf]
        gsems = scratch[2 + 2 * n_buf:2 + 3 * n_buf]
        wsems = scratch[2 + 3 * n_buf:2 + 4 * n_buf]
        wid = lax.axis_index("s") * _NC + lax.axis_index("c")
        batch = wid // tiles_per_batch
        r0 = (wid % tiles_per_batch) * rows_per_tile

        iota16 = lax.iota(jnp.int32, _LANES)

        def issue_gather(c):
            p = c % n_buf
            top = batch * l + (l - 1) - r0 - c * _CHUNK
            gidx = gidxs[p]
            for g in range(_CHUNK // _LANES):
                gidx[pl.ds(g * _LANES, _LANES)] = top - g * _LANES - iota16
            return pltpu.async_copy(x2d.at[gidx], bufs[p], gsems[p])

        gdesc = [issue_gather(c) for c in range(n_buf - 1)] + [None]

        pltpu.sync_copy(tkf.at[pl.ds(batch * num_k, num_k)], tkv)

        for i in range(rows_per_tile // _LANES):
            mask_ref[pl.ds(i * _LANES, _LANES)] = jnp.zeros(
                (_LANES,), jnp.float32)

        def scan_topk(c, _):
            t16 = tkv[pl.ds(c * _LANES, _LANES)]
            rel = t16 - r0
            sel = (rel >= 0) & (rel < rows_per_tile)
            pos = jnp.clip(rel, 0, rows_per_tile - 1)
            plsc.store_scatter(mask_ref, [pos],
                               jnp.ones((_LANES,), jnp.float32), mask=sel)
            return 0

        lax.fori_loop(0, num_k // _LANES, scan_topk, 0)

        zeros16 = jnp.zeros((_LANES,), jnp.float32)

        def zero_unselected(c, buf):
            def zero_row(j, _):
                m = mask_ref[pl.ds(c * _CHUNK + j, _LANES)][0]

                @pl.when(m == 0.0)
                def _():
                    for g in range(vecs_per_row):
                        buf[j, pl.ds(g * _LANES, _LANES)] = zeros16

                return 0

            lax.fori_loop(0, _CHUNK, zero_row, 0)

        wdesc = {}
        waited = set()
        for c in range(n_chunks):
            p = c % n_buf
            gdesc[p].wait()
            zero_unselected(c, bufs[p])
            base = batch * l + r0 + c * _CHUNK
            wdesc[c] = pltpu.async_copy(bufs[p], out2d.at[pl.ds(base, _CHUNK)],
                                        wsems[p])
            nc = c + n_buf - 1
            if nc < n_chunks:
                prev_w = nc - n_buf
                if prev_w >= 0:
                    wdesc[prev_w].wait()
                    waited.add(prev_w)
                gdesc[nc % n_buf] = issue_gather(nc)
        for c in range(n_chunks):
            if c not in waited:
                wdesc[c].wait()

    return pl.kernel(
        body,
        out_type=jax.ShapeDtypeStruct((b * l, d), jnp.float32),
        mesh=mesh,
        compiler_params=pltpu.CompilerParams(needs_layout_passes=False),
        scratch_types=(
            [pltpu.VMEM((num_k,), jnp.int32),
             pltpu.VMEM((rows_per_tile + _LANES,), jnp.float32)]
            + [pltpu.VMEM((_CHUNK,), jnp.int32) for _ in range(5)]
            + [pltpu.VMEM((_CHUNK, d), jnp.float32) for _ in range(5)]
            + [pltpu.SemaphoreType.DMA for _ in range(10)]
        ),
    )


def kernel(x, select_x_mask, router_k, topk_index, Wq, Wk, Wv):
    b, l, d = x.shape
    num_k = topk_index.shape[1]
    x2d = x.reshape(b * l, d)
    tkf = topk_index.astype(jnp.int32).reshape(b * num_k)
    out2d = _sc_masked_rev_copy(b, l, d, num_k)(x2d, tkf)
    return out2d.reshape(b, l, d)


# speedup vs baseline: 1.2264x; 1.0178x over previous
import functools

import jax
import jax.numpy as jnp
from jax import lax
from jax.experimental import pallas as pl
from jax.experimental.pallas import tpu as pltpu
from jax.experimental.pallas import tpu_sc as plsc

_NC = 2
_NS = 16
_LANES = 16
_CHUNK = 32


@functools.lru_cache(maxsize=None)
def _sc_masked_rev_copy(b, l, d, num_k):
    n_workers = _NC * _NS
    tiles_per_batch = n_workers // b
    rows_per_tile = l // tiles_per_batch
    n_chunks = rows_per_tile // _CHUNK
    vecs_per_row = d // _LANES
    mesh = plsc.VectorSubcoreMesh(
        core_axis_name="c", subcore_axis_name="s",
        num_cores=_NC, num_subcores=_NS)

    n_buf = 4

    def body(x2d, tkf, out2d, *scratch):
        tkv, mask_ref = scratch[0], scratch[1]
        gidxs = scratch[2:2 + n_buf]
        bufs = scratch[2 + n_buf:2 + 2 * n_buf]
        gsems = scratch[2 + 2 * n_buf:2 + 3 * n_buf]
        wsems = scratch[2 + 3 * n_buf:2 + 4 * n_buf]
        wid = lax.axis_index("s") * _NC + lax.axis_index("c")
        batch = wid // tiles_per_batch
        r0 = (wid % tiles_per_batch) * rows_per_tile

        iota16 = lax.iota(jnp.int32, _LANES)

        def issue_gather(c):
            p = c % n_buf
            top = batch * l + (l - 1) - r0 - c * _CHUNK
            gidx = gidxs[p]
            for g in range(_CHUNK // _LANES):
                gidx[pl.ds(g * _LANES, _LANES)] = top - g * _LANES - iota16
            return pltpu.async_copy(x2d.at[gidx], bufs[p], gsems[p])

        gdesc = [issue_gather(c) for c in range(n_buf - 1)] + [None]

        pltpu.sync_copy(tkf.at[pl.ds(batch * num_k, num_k)], tkv)

        for i in range(rows_per_tile // _LANES):
            mask_ref[pl.ds(i * _LANES, _LANES)] = jnp.zeros(
                (_LANES,), jnp.float32)

        def scan_topk(c, _):
            t16 = tkv[pl.ds(c * _LANES, _LANES)]
            rel = t16 - r0
            sel = (rel >= 0) & (rel < rows_per_tile)
            pos = jnp.clip(rel, 0, rows_per_tile - 1)
            plsc.store_scatter(mask_ref, [pos],
                               jnp.ones((_LANES,), jnp.float32), mask=sel)
            return 0

        lax.fori_loop(0, num_k // _LANES, scan_topk, 0)

        zeros16 = jnp.zeros((_LANES,), jnp.float32)

        def zero_unselected(c, buf):
            def zero_row(j, _):
                m = mask_ref[pl.ds(c * _CHUNK + j, _LANES)][0]

                @pl.when(m == 0.0)
                def _():
                    for g in range(vecs_per_row):
                        buf[j, pl.ds(g * _LANES, _LANES)] = zeros16

                return 0

            lax.fori_loop(0, _CHUNK, zero_row, 0)

        wdesc = {}
        waited = set()
        for c in range(n_chunks):
            p = c % n_buf
            gdesc[p].wait()
            zero_unselected(c, bufs[p])
            base = batch * l + r0 + c * _CHUNK
            wdesc[c] = pltpu.async_copy(bufs[p], out2d.at[pl.ds(base, _CHUNK)],
                                        wsems[p])
            nc = c + n_buf - 1
            if nc < n_chunks:
                prev_w = nc - n_buf
                if prev_w >= 0:
                    wdesc[prev_w].wait()
                    waited.add(prev_w)
                gdesc[nc % n_buf] = issue_gather(nc)
        for c in range(n_chunks):
            if c not in waited:
                wdesc[c].wait()

    return pl.kernel(
        body,
        out_type=jax.ShapeDtypeStruct((b * l, d), jnp.float32),
        mesh=mesh,
        compiler_params=pltpu.CompilerParams(needs_layout_passes=False),
        scratch_types=(
            [pltpu.VMEM((num_k,), jnp.int32),
             pltpu.VMEM((rows_per_tile + _LANES,), jnp.float32)]
            + [pltpu.VMEM((_CHUNK,), jnp.int32) for _ in range(4)]
            + [pltpu.VMEM((_CHUNK, d), jnp.float32) for _ in range(4)]
            + [pltpu.SemaphoreType.DMA for _ in range(8)]
        ),
    )


def kernel(x, select_x_mask, router_k, topk_index, Wq, Wk, Wv):
    b, l, d = x.shape
    num_k = topk_index.shape[1]
    x2d = x.reshape(b * l, d)
    tkf = topk_index.astype(jnp.int32).reshape(b * num_k)
    out2d = _sc_masked_rev_copy(b, l, d, num_k)(x2d, tkf)
    return out2d.reshape(b, l, d)
